# Initial kernel scaffold; baseline (speedup 1.0000x reference)
import jax
import jax.numpy as jnp
from jax.experimental import pallas as pl


def kernel(input_tokens, base_table, ext_table):
    raise NotImplementedError("write your pallas kernel here")


# speedup vs baseline: 2.3238x; 2.3238x over previous
import functools

import jax
import jax.numpy as jnp
from jax import lax
from jax.experimental import pallas as pl
from jax.experimental.pallas import tpu as pltpu
from jax.experimental.pallas import tpu_sc as plsc

BASE_VOCAB = 1000000
EXT_VOCAB = 1000
EMBED_DIM = 64
THRESHOLD = 1000000

NUM_CORES = 2
NUM_SUBCORES = 16
LANES = 16
NW = NUM_CORES * NUM_SUBCORES

CHUNK = 128


def _body(tok_hbm, base_hbm, ext_hbm, out_hbm, ext_v, tok_v, bidx_v, rows_v, sem):
    n_tokens = tok_hbm.shape[0]
    per_w = n_tokens // NW
    n_chunks = per_w // CHUNK

    wid = lax.axis_index("s") * NUM_CORES + lax.axis_index("c")
    base_off = wid * per_w

    pltpu.sync_copy(ext_hbm, ext_v)

    lanes = lax.broadcasted_iota(jnp.int32, (LANES,), 0)

    @pl.loop(0, n_chunks)
    def _chunk(g):
        off = base_off + g * CHUNK
        pltpu.sync_copy(tok_hbm.at[pl.ds(off, CHUNK)], tok_v)

        for i in range(CHUNK // LANES):
            t = tok_v[pl.ds(i * LANES, LANES)]
            m = t >= THRESHOLD
            bidx_v[pl.ds(i * LANES, LANES)] = jnp.where(m, 0, t)

        pltpu.async_copy(base_hbm.at[bidx_v], rows_v, sem).wait()

        for i in range(CHUNK // LANES):
            t = tok_v[pl.ds(i * LANES, LANES)]
            m = t >= THRESHOLD

            @pl.when(jnp.sum(m.astype(jnp.int32)) > 0)
            def _patch():
                eidx = jnp.where(m, t - THRESHOLD, 0)
                row16 = i * LANES + lanes
                for col in range(EMBED_DIM):
                    col16 = jnp.full((LANES,), col, jnp.int32)
                    vals = plsc.load_gather(ext_v, [eidx, col16], mask=m)
                    plsc.store_scatter(rows_v, [row16, col16], vals, mask=m)

        pltpu.sync_copy(rows_v, out_hbm.at[pl.ds(off, CHUNK)])


@jax.jit
def _run(tok_flat, base_table, ext_table):
    mesh = plsc.VectorSubcoreMesh(
        core_axis_name="c", subcore_axis_name="s",
        num_cores=NUM_CORES, num_subcores=NUM_SUBCORES)
    f = pl.kernel(
        _body,
        out_type=jax.ShapeDtypeStruct((tok_flat.shape[0], EMBED_DIM), jnp.float32),
        mesh=mesh,
        scratch_types=[
            pltpu.VMEM((EXT_VOCAB, EMBED_DIM), jnp.float32),
            pltpu.VMEM((CHUNK,), jnp.int32),
            pltpu.VMEM((CHUNK,), jnp.int32),
            pltpu.VMEM((CHUNK, EMBED_DIM), jnp.float32),
            pltpu.SemaphoreType.DMA,
        ],
        compiler_params=pltpu.CompilerParams(use_tc_tiling_on_sc=False, needs_layout_passes=False),
    )
    return f(tok_flat, base_table, ext_table)


def kernel(input_tokens, base_table, ext_table):
    b, s = input_tokens.shape
    out = _run(input_tokens.reshape(b * s), base_table, ext_table)
    return out.reshape(b, s, EMBED_DIM)
